# baseline (speedup 1.0000x reference)
import functools

import jax
import jax.numpy as jnp
from jax import lax
from jax.experimental import pallas as pl
from jax.experimental.pallas import tpu as pltpu
from jax.experimental.pallas import tpu_sc as plsc

N_NODE = 10000
N_PRICE = 100
N_CB = 20
N_CM = 200
EMB = 128
NNZ = 320000
LAYERS = 2

_NS = 16
_NC = 2
_NP = 2
_EPT = NNZ // _NS
_CH = 80
_NCH = _EPT // _CH
_SEG = N_NODE // (_NC * _NP)
_AR = _SEG + 12
_ZT = 160


def _sc_spmm(row3, col3, val3, item):
    mesh = plsc.VectorSubcoreMesh(core_axis_name="c", subcore_axis_name="s")

    @functools.partial(
        pl.kernel,
        mesh=mesh,
        out_type=jax.ShapeDtypeStruct((_NP, _NC, _AR, EMB), jnp.float32),
        scratch_types=[
            pltpu.VMEM((_NCH, _CH), jnp.int32),
            pltpu.VMEM((_NCH, _CH), jnp.int32),
            pltpu.VMEM((_EPT,), jnp.float32),
            pltpu.VMEM((_CH, EMB), jnp.float32),
            pltpu.VMEM((_CH, EMB), jnp.float32),
            pltpu.VMEM((_CH,), jnp.int32),
            pltpu.VMEM((_CH,), jnp.int32),
            pltpu.VMEM_SHARED((_AR, EMB), jnp.float32),
            pltpu.SemaphoreType.DMA,
            pltpu.SemaphoreType.DMA,
            pltpu.SemaphoreType.DMA,
        ],
    )
    def spmm(row_h, col_h, val_h, item_h, out_h,
             row_v, col_v, val_v, rows_a, rows_b, idx_a, idx_b, acc_s,
             gsem_a, gsem_b, ssem):
        c = lax.axis_index("c")
        s = lax.axis_index("s")
        base = s * _ZT

        pltpu.sync_copy(row_h.at[s], row_v)
        pltpu.sync_copy(col_h.at[s], col_v)
        pltpu.sync_copy(val_h.at[s, 0], val_v)

        for p in range(_NP):
            def zrow(i, carry):
                for q in range(EMB // 16):
                    rows_a[i, pl.ds(q * 16, 16)] = jnp.zeros((16,),
                                                             jnp.float32)
                return carry
            lax.fori_loop(0, _CH, zrow, 0)

            @pl.when(s < _NS - 1)
            def _():
                for q in range(_ZT // _CH):
                    pltpu.sync_copy(rows_a,
                                    acc_s.at[pl.ds(base + q * _CH, _CH)])

            @pl.when(s == _NS - 1)
            def _():
                tail = _AR - (_NS - 1) * _ZT
                pltpu.sync_copy(rows_a.at[pl.ds(0, _CH)],
                                acc_s.at[pl.ds(base, _CH)])
                pltpu.sync_copy(rows_a.at[pl.ds(0, tail - _CH)],
                                acc_s.at[pl.ds(base + _CH, tail - _CH)])
            plsc.subcore_barrier()

            half0 = (_NC * p) * _SEG + c * _SEG

            def scale(j, rows_ref, idx_ref):
                def sgrp(g, c2):
                    vb16 = val_v[pl.ds(j * _CH + g * 16, 16)]
                    r16 = row_v[j, pl.ds(g * 16, 16)]
                    t = r16 - half0
                    inr = (t >= 0) & (t < _SEG)
                    garb = _SEG + (r16 & 7)
                    idx_ref[pl.ds(g * 16, 16)] = jnp.where(inr, t, garb)
                    for r in range(16):
                        tr = t[r]

                        @pl.when((tr >= 0) & (tr < _SEG))
                        def _():
                            vb = jnp.full((16,), vb16[r], jnp.float32)
                            i = g * 16 + r
                            for q in range(EMB // 16):
                                rows_ref[i, pl.ds(q * 16, 16)] = (
                                    rows_ref[i, pl.ds(q * 16, 16)] * vb)
                    return c2
                lax.fori_loop(0, _CH // 16, sgrp, 0)

            pltpu.async_copy(item_h.at[col_v.at[0]], rows_a, gsem_a)
            pltpu.async_copy(item_h.at[col_v.at[1]], rows_b, gsem_b)

            def chunk2(jj, carry):
                j0 = jj * 2
                j1 = j0 + 1

                pltpu.make_async_copy(item_h.at[col_v.at[j0]], rows_a,
                                      gsem_a).wait()
                scale(j0, rows_a, idx_a)
                pltpu.async_copy(rows_a, acc_s.at[idx_a], ssem,
                                 add=True).wait()

                @pl.when(jj < _NCH // 2 - 1)
                def _():
                    pltpu.async_copy(item_h.at[col_v.at[j0 + 2]], rows_a,
                                     gsem_a)

                pltpu.make_async_copy(item_h.at[col_v.at[j1]], rows_b,
                                      gsem_b).wait()
                scale(j1, rows_b, idx_b)
                pltpu.async_copy(rows_b, acc_s.at[idx_b], ssem,
                                 add=True).wait()

                @pl.when(jj < _NCH // 2 - 1)
                def _():
                    pltpu.async_copy(item_h.at[col_v.at[j1 + 2]], rows_b,
                                     gsem_b)
                return carry
            lax.fori_loop(0, _NCH // 2, chunk2, 0)

            plsc.subcore_barrier()

            @pl.when(s < _NS - 1)
            def _():
                pltpu.sync_copy(acc_s.at[pl.ds(base, _ZT)],
                                out_h.at[p, c, pl.ds(base, _ZT)])

            @pl.when(s == _NS - 1)
            def _():
                tail = _AR - (_NS - 1) * _ZT
                pltpu.sync_copy(acc_s.at[pl.ds(base, tail)],
                                out_h.at[p, c, pl.ds(base, tail)])
            plsc.subcore_barrier()

    return spmm(row3, col3, val3, item)


def _intra(adj, mat_v, emb2):
    rows = adj.shape[0]
    mv = jnp.broadcast_to(mat_v, (rows, EMB))
    logits = lax.dot_general(mv, emb2, (((1,), (1,)), ((), ())),
                             preferred_element_type=jnp.float32)
    m = jnp.max(logits, axis=1, keepdims=True)
    e = jnp.exp(logits - m)
    sm = e / jnp.sum(e, axis=1, keepdims=True)
    a = sm * adj
    a = a / (jnp.sum(a, axis=1, keepdims=True) + 1e-8)
    return jnp.dot(a, emb2, preferred_element_type=jnp.float32)


def _inter(W, b, e0, e1, e2, e3):
    x0 = jnp.exp(jnp.dot(e0, W, preferred_element_type=jnp.float32) + b)
    x1 = jnp.exp(jnp.dot(e1, W, preferred_element_type=jnp.float32) + b)
    x2 = jnp.exp(jnp.dot(e2, W, preferred_element_type=jnp.float32) + b)
    x3 = jnp.exp(jnp.dot(e3, W, preferred_element_type=jnp.float32) + b)
    s = x0 + x1 + x2 + x3
    return (x0 / s) * e0 + (x1 / s) * e1 + (x2 / s) * e2 + (x3 / s) * e3


_RB = 1000


def _item_update(item, pri, cb, cm, avp, avcb, avcm,
                 mvp, mvcb, mvcm, Wg, bg, parts):
    def body(item_r, avp_r, avcb_r, avcm_r, mvp_r, mvcb_r, mvcm_r,
             pri_r, cb_r, cm_r, Wg_r, bg_r, parts_r, out_r):
        it = item_r[...]
        hp = _intra(avp_r[...], mvp_r[...], pri_r[...])
        hcb = _intra(avcb_r[...], mvcb_r[...], cb_r[...])
        hcm = _intra(avcm_r[...], mvcm_r[...], cm_r[...])
        g = _inter(Wg_r[...], bg_r[...], it, hp, hcb, hcm)
        out_r[...] = g + parts_r[...]

    return pl.pallas_call(
        body,
        grid=(N_NODE // _RB,),
        in_specs=[
            pl.BlockSpec((_RB, EMB), lambda i: (i, 0)),
            pl.BlockSpec((_RB, N_PRICE), lambda i: (i, 0)),
            pl.BlockSpec((_RB, N_CB), lambda i: (i, 0)),
            pl.BlockSpec((_RB, N_CM), lambda i: (i, 0)),
            pl.BlockSpec((_RB, 1), lambda i: (i, 0)),
            pl.BlockSpec((_RB, 1), lambda i: (i, 0)),
            pl.BlockSpec((_RB, 1), lambda i: (i, 0)),
            pl.BlockSpec((N_PRICE, EMB), lambda i: (0, 0)),
            pl.BlockSpec((N_CB, EMB), lambda i: (0, 0)),
            pl.BlockSpec((N_CM, EMB), lambda i: (0, 0)),
            pl.BlockSpec((EMB, 1), lambda i: (0, 0)),
            pl.BlockSpec((1, 1), lambda i: (0, 0)),
            pl.BlockSpec((_RB, EMB), lambda i: (i, 0)),
        ],
        out_specs=pl.BlockSpec((_RB, EMB), lambda i: (i, 0)),
        out_shape=jax.ShapeDtypeStruct((N_NODE, EMB), jnp.float32),
    )(item, avp, avcb, avcm, mvp, mvcb, mvcm, pri, cb, cm, Wg, bg, parts)


def _small_update(item, pri, cb, cm,
                  apv, apcb, apcm, acbp, acbv, acbcm, acmp, acmv, acmcb,
                  mpv, mpcb, mpcm, mcbp, mcbv, mcbcm, mcmp, mcmv, mcmcb,
                  Wp, bp, Wcb, bcb, Wcm, bcm):
    def body(item_r, pri_r, cb_r, cm_r,
             apv_r, apcb_r, apcm_r, acbp_r, acbv_r, acbcm_r,
             acmp_r, acmv_r, acmcb_r,
             mpv_r, mpcb_r, mpcm_r, mcbp_r, mcbv_r, mcbcm_r,
             mcmp_r, mcmv_r, mcmcb_r,
             Wp_r, bp_r, Wcb_r, bcb_r, Wcm_r, bcm_r,
             pri_o, cb_o, cm_o):
        it = item_r[...]
        p = pri_r[...]
        b_ = cb_r[...]
        m_ = cm_r[...]
        pri_o[...] = _inter(Wp_r[...], bp_r[...], p,
                            _intra(apv_r[...], mpv_r[...], it),
                            _intra(apcb_r[...], mpcb_r[...], b_),
                            _intra(apcm_r[...], mpcm_r[...], m_))
        cb_o[...] = _inter(Wcb_r[...], bcb_r[...], b_,
                           _intra(acbp_r[...], mcbp_r[...], p),
                           _intra(acbv_r[...], mcbv_r[...], it),
                           _intra(acbcm_r[...], mcbcm_r[...], m_))
        cm_o[...] = _inter(Wcm_r[...], bcm_r[...], m_,
                           _intra(acmp_r[...], mcmp_r[...], p),
                           _intra(acmv_r[...], mcmv_r[...], it),
                           _intra(acmcb_r[...], mcmcb_r[...], b_))

    return pl.pallas_call(
        body,
        out_shape=(
            jax.ShapeDtypeStruct((N_PRICE, EMB), jnp.float32),
            jax.ShapeDtypeStruct((N_CB, EMB), jnp.float32),
            jax.ShapeDtypeStruct((N_CM, EMB), jnp.float32),
        ),
    )(item, pri, cb, cm,
      apv, apcb, apcm, acbp, acbv, acbcm, acmp, acmv, acmcb,
      mpv, mpcb, mpcm, mcbp, mcbv, mcbcm, mcmp, mcmv, mcmcb,
      Wp, bp, Wcb, bcb, Wcm, bcm)


def kernel(adjacency_row, adjacency_col, adjacency_val,
           adjacency_vp, adjacency_vcb, adjacency_vcm,
           adjacency_pv, adjacency_pcb, adjacency_pcm,
           adjacency_cbp, adjacency_cbv, adjacency_cbcm,
           adjacency_cmp, adjacency_cmv, adjacency_cmcb,
           item_emb, pri_emb, cateBig_emb, cateMiddle_emb,
           mat_vp, mat_vcb, mat_vcm, mat_pv, mat_pcb, mat_pcm,
           mat_cbp, mat_cbv, mat_cbcm, mat_cmp, mat_cmv, mat_cmcb,
           W_gi, b_gi, W_gp, b_gp, W_gcb, b_gcb, W_gcm, b_gcm):
    row3 = adjacency_row.reshape(_NS, _NCH, _CH)
    col3 = adjacency_col.reshape(_NS, _NCH, _CH)
    val3 = adjacency_val.reshape(_NS, 1, _EPT)
    bgi = b_gi.reshape(1, 1)
    bgp = b_gp.reshape(1, 1)
    bgcb = b_gcb.reshape(1, 1)
    bgcm = b_gcm.reshape(1, 1)

    item, pri, cb, cm = item_emb, pri_emb, cateBig_emb, cateMiddle_emb
    for _ in range(LAYERS):
        seg = _sc_spmm(row3, col3, val3, item)
        parts = jnp.concatenate(
            [seg[0, 0, :_SEG], seg[0, 1, :_SEG],
             seg[1, 0, :_SEG], seg[1, 1, :_SEG]], axis=0)
        item_n = _item_update(item, pri, cb, cm,
                              adjacency_vp, adjacency_vcb, adjacency_vcm,
                              mat_vp, mat_vcb, mat_vcm, W_gi, bgi, parts)
        pri_n, cb_n, cm_n = _small_update(
            item, pri, cb, cm,
            adjacency_pv, adjacency_pcb, adjacency_pcm,
            adjacency_cbp, adjacency_cbv, adjacency_cbcm,
            adjacency_cmp, adjacency_cmv, adjacency_cmcb,
            mat_pv, mat_pcb, mat_pcm, mat_cbp, mat_cbv, mat_cbcm,
            mat_cmp, mat_cmv, mat_cmcb,
            W_gp, bgp, W_gcb, bgcb, W_gcm, bgcm)
        item, pri, cb, cm = item_n, pri_n, cb_n, cm_n
    return (item, pri)


# speedup vs baseline: 1.1234x; 1.1234x over previous
import functools

import jax
import jax.numpy as jnp
from jax import lax
from jax.experimental import pallas as pl
from jax.experimental.pallas import tpu as pltpu
from jax.experimental.pallas import tpu_sc as plsc

N_NODE = 10000
N_PRICE = 100
N_CB = 20
N_CM = 200
EMB = 128
NNZ = 320000
LAYERS = 2

_NS = 16
_NC = 2
_NP = 2
_EPT = NNZ // _NS
_CH = 80
_NCH = _EPT // _CH
_SEG = N_NODE // (_NC * _NP)
_AR = _SEG + 12
_ZT = 160


def _sc_spmm(row3, col3, val3, item):
    mesh = plsc.VectorSubcoreMesh(core_axis_name="c", subcore_axis_name="s")

    @functools.partial(
        pl.kernel,
        mesh=mesh,
        out_type=jax.ShapeDtypeStruct((_NP, _NC, _AR, EMB), jnp.float32),
        scratch_types=[
            pltpu.VMEM((_NCH, _CH), jnp.int32),
            pltpu.VMEM((_NCH, _CH), jnp.int32),
            pltpu.VMEM((_EPT,), jnp.float32),
            pltpu.VMEM((_CH, EMB), jnp.float32),
            pltpu.VMEM((_CH, EMB), jnp.float32),
            pltpu.VMEM((_CH,), jnp.int32),
            pltpu.VMEM((_CH,), jnp.int32),
            pltpu.VMEM_SHARED((_AR, EMB), jnp.float32),
            pltpu.SemaphoreType.DMA,
            pltpu.SemaphoreType.DMA,
            pltpu.SemaphoreType.DMA,
            pltpu.SemaphoreType.DMA,
        ],
    )
    def spmm(row_h, col_h, val_h, item_h, out_h,
             row_v, col_v, val_v, rows_a, rows_b, idx_a, idx_b, acc_s,
             gsem_a, gsem_b, ssem_a, ssem_b):
        c = lax.axis_index("c")
        s = lax.axis_index("s")
        base = s * _ZT

        pltpu.sync_copy(row_h.at[s], row_v)
        pltpu.sync_copy(col_h.at[s], col_v)
        pltpu.sync_copy(val_h.at[s, 0], val_v)

        for p in range(_NP):
            def zrow(i, carry):
                for q in range(EMB // 16):
                    rows_a[i, pl.ds(q * 16, 16)] = jnp.zeros((16,),
                                                             jnp.float32)
                return carry
            lax.fori_loop(0, _CH, zrow, 0)

            @pl.when(s < _NS - 1)
            def _():
                for q in range(_ZT // _CH):
                    pltpu.sync_copy(rows_a,
                                    acc_s.at[pl.ds(base + q * _CH, _CH)])

            @pl.when(s == _NS - 1)
            def _():
                tail = _AR - (_NS - 1) * _ZT
                pltpu.sync_copy(rows_a.at[pl.ds(0, _CH)],
                                acc_s.at[pl.ds(base, _CH)])
                pltpu.sync_copy(rows_a.at[pl.ds(0, tail - _CH)],
                                acc_s.at[pl.ds(base + _CH, tail - _CH)])
            plsc.subcore_barrier()

            half0 = (_NC * p) * _SEG + c * _SEG

            def scale(j, rows_ref, idx_ref):
                def sgrp(g, c2):
                    vb16 = val_v[pl.ds(j * _CH + g * 16, 16)]
                    r16 = row_v[j, pl.ds(g * 16, 16)]
                    t = r16 - half0
                    inr = (t >= 0) & (t < _SEG)
                    garb = _SEG + (r16 & 7)
                    idx_ref[pl.ds(g * 16, 16)] = jnp.where(inr, t, garb)
                    for r in range(16):
                        vb = jnp.full((16,), vb16[r], jnp.float32)
                        i = g * 16 + r
                        for q in range(EMB // 16):
                            rows_ref[i, pl.ds(q * 16, 16)] = (
                                rows_ref[i, pl.ds(q * 16, 16)] * vb)
                    return c2
                lax.fori_loop(0, _CH // 16, sgrp, 0)

            pltpu.async_copy(item_h.at[col_v.at[0]], rows_a, gsem_a)
            pltpu.async_copy(item_h.at[col_v.at[1]], rows_b, gsem_b)

            def chunk2(jj, carry):
                j0 = jj * 2
                j1 = j0 + 1

                pltpu.make_async_copy(item_h.at[col_v.at[j0]], rows_a,
                                      gsem_a).wait()
                scale(j0, rows_a, idx_a)
                pltpu.async_copy(rows_a, acc_s.at[idx_a], ssem_a, add=True)

                pltpu.make_async_copy(item_h.at[col_v.at[j1]], rows_b,
                                      gsem_b).wait()
                scale(j1, rows_b, idx_b)

                pltpu.make_async_copy(rows_a, acc_s.at[idx_a],
                                      ssem_a).wait()

                @pl.when(jj < _NCH // 2 - 1)
                def _():
                    pltpu.async_copy(item_h.at[col_v.at[j0 + 2]], rows_a,
                                     gsem_a)

                pltpu.async_copy(rows_b, acc_s.at[idx_b], ssem_b,
                                 add=True).wait()

                @pl.when(jj < _NCH // 2 - 1)
                def _():
                    pltpu.async_copy(item_h.at[col_v.at[j1 + 2]], rows_b,
                                     gsem_b)
                return carry
            lax.fori_loop(0, _NCH // 2, chunk2, 0)

            plsc.subcore_barrier()

            @pl.when(s < _NS - 1)
            def _():
                pltpu.sync_copy(acc_s.at[pl.ds(base, _ZT)],
                                out_h.at[p, c, pl.ds(base, _ZT)])

            @pl.when(s == _NS - 1)
            def _():
                tail = _AR - (_NS - 1) * _ZT
                pltpu.sync_copy(acc_s.at[pl.ds(base, tail)],
                                out_h.at[p, c, pl.ds(base, tail)])
            plsc.subcore_barrier()

    return spmm(row3, col3, val3, item)


def _intra(adj, mat_v, emb2):
    rows = adj.shape[0]
    mv = jnp.broadcast_to(mat_v, (rows, EMB))
    logits = lax.dot_general(mv, emb2, (((1,), (1,)), ((), ())),
                             preferred_element_type=jnp.float32)
    m = jnp.max(logits, axis=1, keepdims=True)
    e = jnp.exp(logits - m)
    sm = e / jnp.sum(e, axis=1, keepdims=True)
    a = sm * adj
    a = a / (jnp.sum(a, axis=1, keepdims=True) + 1e-8)
    return jnp.dot(a, emb2, preferred_element_type=jnp.float32)


def _inter(W, b, e0, e1, e2, e3):
    x0 = jnp.exp(jnp.dot(e0, W, preferred_element_type=jnp.float32) + b)
    x1 = jnp.exp(jnp.dot(e1, W, preferred_element_type=jnp.float32) + b)
    x2 = jnp.exp(jnp.dot(e2, W, preferred_element_type=jnp.float32) + b)
    x3 = jnp.exp(jnp.dot(e3, W, preferred_element_type=jnp.float32) + b)
    s = x0 + x1 + x2 + x3
    return (x0 / s) * e0 + (x1 / s) * e1 + (x2 / s) * e2 + (x3 / s) * e3


_RB = 1000


def _item_update(item, pri, cb, cm, avp, avcb, avcm,
                 mvp, mvcb, mvcm, Wg, bg, parts):
    def body(item_r, avp_r, avcb_r, avcm_r, mvp_r, mvcb_r, mvcm_r,
             pri_r, cb_r, cm_r, Wg_r, bg_r, parts_r, out_r):
        it = item_r[...]
        hp = _intra(avp_r[...], mvp_r[...], pri_r[...])
        hcb = _intra(avcb_r[...], mvcb_r[...], cb_r[...])
        hcm = _intra(avcm_r[...], mvcm_r[...], cm_r[...])
        g = _inter(Wg_r[...], bg_r[...], it, hp, hcb, hcm)
        out_r[...] = g + parts_r[...]

    return pl.pallas_call(
        body,
        grid=(N_NODE // _RB,),
        in_specs=[
            pl.BlockSpec((_RB, EMB), lambda i: (i, 0)),
            pl.BlockSpec((_RB, N_PRICE), lambda i: (i, 0)),
            pl.BlockSpec((_RB, N_CB), lambda i: (i, 0)),
            pl.BlockSpec((_RB, N_CM), lambda i: (i, 0)),
            pl.BlockSpec((_RB, 1), lambda i: (i, 0)),
            pl.BlockSpec((_RB, 1), lambda i: (i, 0)),
            pl.BlockSpec((_RB, 1), lambda i: (i, 0)),
            pl.BlockSpec((N_PRICE, EMB), lambda i: (0, 0)),
            pl.BlockSpec((N_CB, EMB), lambda i: (0, 0)),
            pl.BlockSpec((N_CM, EMB), lambda i: (0, 0)),
            pl.BlockSpec((EMB, 1), lambda i: (0, 0)),
            pl.BlockSpec((1, 1), lambda i: (0, 0)),
            pl.BlockSpec((_RB, EMB), lambda i: (i, 0)),
        ],
        out_specs=pl.BlockSpec((_RB, EMB), lambda i: (i, 0)),
        out_shape=jax.ShapeDtypeStruct((N_NODE, EMB), jnp.float32),
    )(item, avp, avcb, avcm, mvp, mvcb, mvcm, pri, cb, cm, Wg, bg, parts)


def _small_update(item, pri, cb, cm,
                  apv, apcb, apcm, acbp, acbv, acbcm, acmp, acmv, acmcb,
                  mpv, mpcb, mpcm, mcbp, mcbv, mcbcm, mcmp, mcmv, mcmcb,
                  Wp, bp, Wcb, bcb, Wcm, bcm):
    def body(item_r, pri_r, cb_r, cm_r,
             apv_r, apcb_r, apcm_r, acbp_r, acbv_r, acbcm_r,
             acmp_r, acmv_r, acmcb_r,
             mpv_r, mpcb_r, mpcm_r, mcbp_r, mcbv_r, mcbcm_r,
             mcmp_r, mcmv_r, mcmcb_r,
             Wp_r, bp_r, Wcb_r, bcb_r, Wcm_r, bcm_r,
             pri_o, cb_o, cm_o):
        it = item_r[...]
        p = pri_r[...]
        b_ = cb_r[...]
        m_ = cm_r[...]
        pri_o[...] = _inter(Wp_r[...], bp_r[...], p,
                            _intra(apv_r[...], mpv_r[...], it),
                            _intra(apcb_r[...], mpcb_r[...], b_),
                            _intra(apcm_r[...], mpcm_r[...], m_))
        cb_o[...] = _inter(Wcb_r[...], bcb_r[...], b_,
                           _intra(acbp_r[...], mcbp_r[...], p),
                           _intra(acbv_r[...], mcbv_r[...], it),
                           _intra(acbcm_r[...], mcbcm_r[...], m_))
        cm_o[...] = _inter(Wcm_r[...], bcm_r[...], m_,
                           _intra(acmp_r[...], mcmp_r[...], p),
                           _intra(acmv_r[...], mcmv_r[...], it),
                           _intra(acmcb_r[...], mcmcb_r[...], b_))

    return pl.pallas_call(
        body,
        out_shape=(
            jax.ShapeDtypeStruct((N_PRICE, EMB), jnp.float32),
            jax.ShapeDtypeStruct((N_CB, EMB), jnp.float32),
            jax.ShapeDtypeStruct((N_CM, EMB), jnp.float32),
        ),
    )(item, pri, cb, cm,
      apv, apcb, apcm, acbp, acbv, acbcm, acmp, acmv, acmcb,
      mpv, mpcb, mpcm, mcbp, mcbv, mcbcm, mcmp, mcmv, mcmcb,
      Wp, bp, Wcb, bcb, Wcm, bcm)


def kernel(adjacency_row, adjacency_col, adjacency_val,
           adjacency_vp, adjacency_vcb, adjacency_vcm,
           adjacency_pv, adjacency_pcb, adjacency_pcm,
           adjacency_cbp, adjacency_cbv, adjacency_cbcm,
           adjacency_cmp, adjacency_cmv, adjacency_cmcb,
           item_emb, pri_emb, cateBig_emb, cateMiddle_emb,
           mat_vp, mat_vcb, mat_vcm, mat_pv, mat_pcb, mat_pcm,
           mat_cbp, mat_cbv, mat_cbcm, mat_cmp, mat_cmv, mat_cmcb,
           W_gi, b_gi, W_gp, b_gp, W_gcb, b_gcb, W_gcm, b_gcm):
    row3 = adjacency_row.reshape(_NS, _NCH, _CH)
    col3 = adjacency_col.reshape(_NS, _NCH, _CH)
    val3 = adjacency_val.reshape(_NS, 1, _EPT)
    bgi = b_gi.reshape(1, 1)
    bgp = b_gp.reshape(1, 1)
    bgcb = b_gcb.reshape(1, 1)
    bgcm = b_gcm.reshape(1, 1)

    item, pri, cb, cm = item_emb, pri_emb, cateBig_emb, cateMiddle_emb
    for _ in range(LAYERS):
        seg = _sc_spmm(row3, col3, val3, item)
        parts = jnp.concatenate(
            [seg[0, 0, :_SEG], seg[0, 1, :_SEG],
             seg[1, 0, :_SEG], seg[1, 1, :_SEG]], axis=0)
        item_n = _item_update(item, pri, cb, cm,
                              adjacency_vp, adjacency_vcb, adjacency_vcm,
                              mat_vp, mat_vcb, mat_vcm, W_gi, bgi, parts)
        pri_n, cb_n, cm_n = _small_update(
            item, pri, cb, cm,
            adjacency_pv, adjacency_pcb, adjacency_pcm,
            adjacency_cbp, adjacency_cbv, adjacency_cbcm,
            adjacency_cmp, adjacency_cmv, adjacency_cmcb,
            mat_pv, mat_pcb, mat_pcm, mat_cbp, mat_cbv, mat_cbcm,
            mat_cmp, mat_cmv, mat_cmcb,
            W_gp, bgp, W_gcb, bgcb, W_gcm, bgcm)
        item, pri, cb, cm = item_n, pri_n, cb_n, cm_n
    return (item, pri)


# speedup vs baseline: 1.2046x; 1.0722x over previous
import functools

import jax
import jax.numpy as jnp
from jax import lax
from jax.experimental import pallas as pl
from jax.experimental.pallas import tpu as pltpu
from jax.experimental.pallas import tpu_sc as plsc

N_NODE = 10000
N_PRICE = 100
N_CB = 20
N_CM = 200
EMB = 128
NNZ = 320000
LAYERS = 2

_NS = 16
_NC = 2
_NP = 2
_EPT = NNZ // _NS
_CH = 80
_NCH = _EPT // _CH
_SEG = 2504
_AR = _SEG + 8
_ZT = 160
_WT = 160
_SEGM = 26802


def _sc_spmm(row3, col3, val3, item):
    mesh = plsc.VectorSubcoreMesh(core_axis_name="c", subcore_axis_name="s")

    @functools.partial(
        pl.kernel,
        mesh=mesh,
        out_type=jax.ShapeDtypeStruct((_NC * _NP * _SEG, EMB), jnp.float32),
        scratch_types=[
            pltpu.VMEM((_NCH, _CH), jnp.int32),
            pltpu.VMEM((_NCH, _CH), jnp.int32),
            pltpu.VMEM((_EPT,), jnp.float32),
            pltpu.VMEM((_CH, EMB), jnp.float32),
            pltpu.VMEM((_CH, EMB), jnp.float32),
            pltpu.VMEM((_CH,), jnp.int32),
            pltpu.VMEM((_CH,), jnp.int32),
            pltpu.VMEM_SHARED((_AR, EMB), jnp.float32),
            pltpu.SemaphoreType.DMA,
            pltpu.SemaphoreType.DMA,
            pltpu.SemaphoreType.DMA,
        ],
    )
    def spmm(row_h, col_h, val_h, item_h, out_h,
             row_v, col_v, val_v, rows_a, rows_b, idx_a, idx_b, acc_s,
             gsem_a, gsem_b, ssem):
        c = lax.axis_index("c")
        s = lax.axis_index("s")
        base = s * _ZT

        pltpu.sync_copy(row_h.at[s], row_v)
        pltpu.sync_copy(col_h.at[s], col_v)
        pltpu.sync_copy(val_h.at[s, 0], val_v)

        for p in range(_NP):
            def zrow(i, carry):
                for q in range(EMB // 16):
                    rows_a[i, pl.ds(q * 16, 16)] = jnp.zeros((16,),
                                                             jnp.float32)
                return carry
            lax.fori_loop(0, _CH, zrow, 0)

            @pl.when(s < _NS - 1)
            def _():
                for q in range(_ZT // _CH):
                    pltpu.sync_copy(rows_a,
                                    acc_s.at[pl.ds(base + q * _CH, _CH)])

            @pl.when(s == _NS - 1)
            def _():
                tail = _AR - (_NS - 1) * _ZT
                pltpu.sync_copy(rows_a.at[pl.ds(0, _CH)],
                                acc_s.at[pl.ds(base, _CH)])
                pltpu.sync_copy(rows_a.at[pl.ds(0, tail - _CH)],
                                acc_s.at[pl.ds(base + _CH, tail - _CH)])
            plsc.subcore_barrier()

            half0 = ((_NC * p) + c) * _SEG

            def scale(j, rows_ref, idx_ref):
                def sgrp(g, c2):
                    vb16 = val_v[pl.ds(j * _CH + g * 16, 16)]
                    r16 = row_v[j, pl.ds(g * 16, 16)]
                    t = r16 - half0
                    inr = (t >= 0) & (t < _SEG)
                    garb = _SEG + (r16 & 7)
                    idx_ref[pl.ds(g * 16, 16)] = jnp.where(inr, t, garb)
                    for r in range(16):
                        vb = jnp.full((16,), vb16[r], jnp.float32)
                        i = g * 16 + r
                        for q in range(EMB // 16):
                            rows_ref[i, pl.ds(q * 16, 16)] = (
                                rows_ref[i, pl.ds(q * 16, 16)] * vb)
                    return c2
                lax.fori_loop(0, _CH // 16, sgrp, 0)

            pltpu.async_copy(item_h.at[col_v.at[0]], rows_a, gsem_a)
            pltpu.async_copy(item_h.at[col_v.at[1]], rows_b, gsem_b)

            def chunk2(jj, carry):
                j0 = jj * 2
                j1 = j0 + 1

                pltpu.make_async_copy(item_h.at[col_v.at[j0]], rows_a,
                                      gsem_a).wait()
                scale(j0, rows_a, idx_a)
                pltpu.async_copy(rows_a, acc_s.at[idx_a], ssem,
                                 add=True).wait()

                @pl.when(jj < _NCH // 2 - 1)
                def _():
                    pltpu.async_copy(item_h.at[col_v.at[j0 + 2]], rows_a,
                                     gsem_a)

                pltpu.make_async_copy(item_h.at[col_v.at[j1]], rows_b,
                                      gsem_b).wait()
                scale(j1, rows_b, idx_b)
                pltpu.async_copy(rows_b, acc_s.at[idx_b], ssem,
                                 add=True).wait()

                @pl.when(jj < _NCH // 2 - 1)
                def _():
                    pltpu.async_copy(item_h.at[col_v.at[j1 + 2]], rows_b,
                                     gsem_b)
                return carry
            lax.fori_loop(0, _NCH // 2, chunk2, 0)

            plsc.subcore_barrier()

            @pl.when(s < _NS - 1)
            def _():
                pltpu.sync_copy(acc_s.at[pl.ds(base, _WT)],
                                out_h.at[pl.ds(half0 + base, _WT)])

            @pl.when(s == _NS - 1)
            def _():
                tail = _SEG - (_NS - 1) * _WT
                pltpu.sync_copy(acc_s.at[pl.ds(base, tail)],
                                out_h.at[pl.ds(half0 + base, tail)])
            plsc.subcore_barrier()

    return spmm(row3, col3, val3, item)


def _intra(adj, mat_v, emb2):
    rows = adj.shape[0]
    mv = jnp.broadcast_to(mat_v, (rows, EMB))
    logits = lax.dot_general(mv, emb2, (((1,), (1,)), ((), ())),
                             preferred_element_type=jnp.float32)
    m = jnp.max(logits, axis=1, keepdims=True)
    e = jnp.exp(logits - m)
    sm = e / jnp.sum(e, axis=1, keepdims=True)
    a = sm * adj
    a = a / (jnp.sum(a, axis=1, keepdims=True) + 1e-8)
    return jnp.dot(a, emb2, preferred_element_type=jnp.float32)


def _inter(W, b, e0, e1, e2, e3):
    x0 = jnp.exp(jnp.dot(e0, W, preferred_element_type=jnp.float32) + b)
    x1 = jnp.exp(jnp.dot(e1, W, preferred_element_type=jnp.float32) + b)
    x2 = jnp.exp(jnp.dot(e2, W, preferred_element_type=jnp.float32) + b)
    x3 = jnp.exp(jnp.dot(e3, W, preferred_element_type=jnp.float32) + b)
    s = x0 + x1 + x2 + x3
    return (x0 / s) * e0 + (x1 / s) * e1 + (x2 / s) * e2 + (x3 / s) * e3


_RB = 1000


def _item_update(item, pri, cb, cm, avp, avcb, avcm,
                 mvp, mvcb, mvcm, Wg, bg, parts):
    def body(item_r, avp_r, avcb_r, avcm_r, mvp_r, mvcb_r, mvcm_r,
             pri_r, cb_r, cm_r, Wg_r, bg_r, parts_r, out_r):
        it = item_r[...]
        hp = _intra(avp_r[...], mvp_r[...], pri_r[...])
        hcb = _intra(avcb_r[...], mvcb_r[...], cb_r[...])
        hcm = _intra(avcm_r[...], mvcm_r[...], cm_r[...])
        g = _inter(Wg_r[...], bg_r[...], it, hp, hcb, hcm)
        out_r[...] = g + parts_r[...]

    return pl.pallas_call(
        body,
        grid=(N_NODE // _RB,),
        in_specs=[
            pl.BlockSpec((_RB, EMB), lambda i: (i, 0)),
            pl.BlockSpec((_RB, N_PRICE), lambda i: (i, 0)),
            pl.BlockSpec((_RB, N_CB), lambda i: (i, 0)),
            pl.BlockSpec((_RB, N_CM), lambda i: (i, 0)),
            pl.BlockSpec((_RB, 1), lambda i: (i, 0)),
            pl.BlockSpec((_RB, 1), lambda i: (i, 0)),
            pl.BlockSpec((_RB, 1), lambda i: (i, 0)),
            pl.BlockSpec((N_PRICE, EMB), lambda i: (0, 0)),
            pl.BlockSpec((N_CB, EMB), lambda i: (0, 0)),
            pl.BlockSpec((N_CM, EMB), lambda i: (0, 0)),
            pl.BlockSpec((EMB, 1), lambda i: (0, 0)),
            pl.BlockSpec((1, 1), lambda i: (0, 0)),
            pl.BlockSpec((_RB, EMB), lambda i: (i, 0)),
        ],
        out_specs=pl.BlockSpec((_RB, EMB), lambda i: (i, 0)),
        out_shape=jax.ShapeDtypeStruct((N_NODE, EMB), jnp.float32),
    )(item, avp, avcb, avcm, mvp, mvcb, mvcm, pri, cb, cm, Wg, bg, parts)


def _small_update(item, pri, cb, cm,
                  apv, apcb, apcm, acbp, acbv, acbcm, acmp, acmv, acmcb,
                  mpv, mpcb, mpcm, mcbp, mcbv, mcbcm, mcmp, mcmv, mcmcb,
                  Wp, bp, Wcb, bcb, Wcm, bcm):
    def body(item_r, pri_r, cb_r, cm_r,
             apv_r, apcb_r, apcm_r, acbp_r, acbv_r, acbcm_r,
             acmp_r, acmv_r, acmcb_r,
             mpv_r, mpcb_r, mpcm_r, mcbp_r, mcbv_r, mcbcm_r,
             mcmp_r, mcmv_r, mcmcb_r,
             Wp_r, bp_r, Wcb_r, bcb_r, Wcm_r, bcm_r,
             pri_o, cb_o, cm_o):
        it = item_r[...]
        p = pri_r[...]
        b_ = cb_r[...]
        m_ = cm_r[...]
        pri_o[...] = _inter(Wp_r[...], bp_r[...], p,
                            _intra(apv_r[...], mpv_r[...], it),
                            _intra(apcb_r[...], mpcb_r[...], b_),
                            _intra(apcm_r[...], mpcm_r[...], m_))
        cb_o[...] = _inter(Wcb_r[...], bcb_r[...], b_,
                           _intra(acbp_r[...], mcbp_r[...], p),
                           _intra(acbv_r[...], mcbv_r[...], it),
                           _intra(acbcm_r[...], mcbcm_r[...], m_))
        cm_o[...] = _inter(Wcm_r[...], bcm_r[...], m_,
                           _intra(acmp_r[...], mcmp_r[...], p),
                           _intra(acmv_r[...], mcmv_r[...], it),
                           _intra(acmcb_r[...], mcmcb_r[...], b_))

    return pl.pallas_call(
        body,
        out_shape=(
            jax.ShapeDtypeStruct((N_PRICE, EMB), jnp.float32),
            jax.ShapeDtypeStruct((N_CB, EMB), jnp.float32),
            jax.ShapeDtypeStruct((N_CM, EMB), jnp.float32),
        ),
    )(item, pri, cb, cm,
      apv, apcb, apcm, acbp, acbv, acbcm, acmp, acmv, acmcb,
      mpv, mpcb, mpcm, mcbp, mcbv, mcbcm, mcmp, mcmv, mcmcb,
      Wp, bp, Wcb, bcb, Wcm, bcm)


def kernel(adjacency_row, adjacency_col, adjacency_val,
           adjacency_vp, adjacency_vcb, adjacency_vcm,
           adjacency_pv, adjacency_pcb, adjacency_pcm,
           adjacency_cbp, adjacency_cbv, adjacency_cbcm,
           adjacency_cmp, adjacency_cmv, adjacency_cmcb,
           item_emb, pri_emb, cateBig_emb, cateMiddle_emb,
           mat_vp, mat_vcb, mat_vcm, mat_pv, mat_pcb, mat_pcm,
           mat_cbp, mat_cbv, mat_cbcm, mat_cmp, mat_cmv, mat_cmcb,
           W_gi, b_gi, W_gp, b_gp, W_gcb, b_gcb, W_gcm, b_gcm):
    row3 = adjacency_row.reshape(_NS, _NCH, _CH)
    col3 = adjacency_col.reshape(_NS, _NCH, _CH)
    val3 = adjacency_val.reshape(_NS, 1, _EPT)
    bgi = b_gi.reshape(1, 1)
    bgp = b_gp.reshape(1, 1)
    bgcb = b_gcb.reshape(1, 1)
    bgcm = b_gcm.reshape(1, 1)

    item, pri, cb, cm = item_emb, pri_emb, cateBig_emb, cateMiddle_emb
    for _ in range(LAYERS):
        parts = _sc_spmm(row3, col3, val3, item)
        item_n = _item_update(item, pri, cb, cm,
                              adjacency_vp, adjacency_vcb, adjacency_vcm,
                              mat_vp, mat_vcb, mat_vcm, W_gi, bgi, parts)
        pri_n, cb_n, cm_n = _small_update(
            item, pri, cb, cm,
            adjacency_pv, adjacency_pcb, adjacency_pcm,
            adjacency_cbp, adjacency_cbv, adjacency_cbcm,
            adjacency_cmp, adjacency_cmv, adjacency_cmcb,
            mat_pv, mat_pcb, mat_pcm, mat_cbp, mat_cbv, mat_cbcm,
            mat_cmp, mat_cmv, mat_cmcb,
            W_gp, bgp, W_gcb, bgcb, W_gcm, bgcm)
        item, pri, cb, cm = item_n, pri_n, cb_n, cm_n
    return (item, pri)
